# baseline (speedup 1.0000x reference)
import jax
import jax.numpy as jnp
from jax import lax
from jax.experimental import pallas as pl
from jax.experimental.pallas import tpu as pltpu

_NUM_LABELS = 81
_POS_K = 64
_NEG_K = 192
_TILE = 1024
_B, _N, _M = 4, 20000, 100
_NP = ((_N + _TILE - 1) // _TILE) * _TILE
_NT = _NP // _TILE


def _make_priorities():
    kp, kn = jax.random.split(jax.random.key(42))
    r_pos = jax.random.randint(kp, (_B, _N), 1, _POS_K * 10, dtype=jnp.int32)
    r_neg = jax.random.randint(kn, (_B, _N), 1, _NEG_K * 10, dtype=jnp.int32)
    pad = ((0, 0), (0, _NP - _N))
    return (jnp.pad(r_pos, pad).reshape(_B, _NT, 1, _TILE),
            jnp.pad(r_neg, pad).reshape(_B, _NT, 1, _TILE))


def _iou_gather_kernel(roi_ref, gt_ref, gv_ref, rp_ref, rn_ref,
                       mp_ref, mn_ref, pack_ref):
    r = roi_ref[0, 0]
    by1 = r[0:1, :]
    bx1 = r[1:2, :]
    by2 = r[2:3, :]
    bx2 = r[3:4, :]
    g = gt_ref[0]
    gy1 = g[:, 0:1]
    gx1 = g[:, 1:2]
    gy2 = g[:, 2:3]
    gx2 = g[:, 3:4]
    x_top = jnp.maximum(bx1, gx1)
    y_top = jnp.maximum(by1, gy1)
    x_bot = jnp.minimum(bx2, gx2)
    y_bot = jnp.minimum(by2, gy2)
    inter = jnp.maximum(x_bot - x_top, 0.0) * jnp.maximum(y_bot - y_top, 0.0)
    barea = (by2 - by1) * (bx2 - bx1)
    iou = inter / (barea - inter)
    m = iou.shape[0]
    mx = jnp.max(iou, axis=0, keepdims=True)
    eqm = iou == mx
    i0 = lax.broadcasted_iota(jnp.int32, iou.shape, 0)
    am = jnp.min(jnp.where(eqm, i0, m), axis=0, keepdims=True)
    onehot = (i0 == am).astype(jnp.float32)
    pack = lax.dot_general(gv_ref[0], onehot, (((1,), (0,)), ((), ())),
                           precision=lax.Precision.HIGHEST,
                           preferred_element_type=jnp.float32)
    pos_c = mx > 0.5
    neg_c = jnp.logical_and(mx < 0.5, mx > 0.1)
    mp_ref[0, 0] = jnp.where(pos_c, rp_ref[0, 0], 0)
    mn_ref[0, 0] = jnp.where(neg_c, rn_ref[0, 0], 0)
    pack_ref[0, 0] = pack


def _select_thresholds(mfull, gidx, k_sel, hi0, n):
    def cnt_gt(t):
        return jnp.sum((mfull > t).astype(jnp.int32))

    def body_t(_, c):
        lo, hi = c
        mid = (lo + hi) // 2
        pred = cnt_gt(mid) < k_sel
        return (jnp.where(pred, lo, mid + 1), jnp.where(pred, mid, hi))

    lo, _ = lax.fori_loop(0, 11, body_t, (jnp.int32(0), jnp.int32(hi0)))
    t_val = lo
    need = k_sel - cnt_gt(t_val)
    eq = mfull == t_val

    def cnt_eq_lt(i):
        return jnp.sum(jnp.where(jnp.logical_and(eq, gidx < i), 1, 0))

    def body_i(_, c):
        lo2, hi2 = c
        mid = (lo2 + hi2) // 2
        pred = cnt_eq_lt(mid) >= need
        return (jnp.where(pred, lo2, mid + 1), jnp.where(pred, mid, hi2))

    lo2, _ = lax.fori_loop(0, 15, body_i, (jnp.int32(0), jnp.int32(n)))
    return t_val, lo2


def _assign_kernel(roi_ref, mp_ref, mn_ref, pack_ref, mpf_ref, mnf_ref,
                   lout_ref, dout_ref, sm_ref):
    k = pl.program_id(0)

    @pl.when(k == 0)
    def _():
        mpf = mpf_ref[...]
        mnf = mnf_ref[...]
        nb, nt, _, tile = mpf.shape
        n = nt * tile
        gidx = (lax.broadcasted_iota(jnp.int32, mpf.shape, 1) * tile
                + lax.broadcasted_iota(jnp.int32, mpf.shape, 3))
        for bi in range(nb):
            t_p, i_p = _select_thresholds(mpf[bi], gidx[bi], _POS_K,
                                          _POS_K * 10 - 1, n)
            t_n, i_n = _select_thresholds(mnf[bi], gidx[bi], _NEG_K,
                                          _NEG_K * 10 - 1, n)
            sm_ref[4 * bi + 0] = t_p
            sm_ref[4 * bi + 1] = i_p
            sm_ref[4 * bi + 2] = t_n
            sm_ref[4 * bi + 3] = i_n

    mp = mp_ref[...][:, 0, 0, :]
    mn = mn_ref[...][:, 0, 0, :]
    b, tile = mp.shape
    i32 = jnp.int32

    def col(off):
        return jnp.concatenate(
            [jnp.full((1, 1), sm_ref[4 * bi + off], i32) for bi in range(b)],
            axis=0)

    t_p = col(0)
    i_p = col(1)
    t_n = col(2)
    i_n = col(3)
    gidx = k * tile + lax.broadcasted_iota(jnp.int32, (b, tile), 1)
    sel_p = (mp > t_p) & (mp > 0) | ((mp == t_p) & (gidx < i_p) & (mp > 0))
    sel_n = (mn > t_n) & (mn > 0) | ((mn == t_n) & (gidx < i_n) & (mn > 0))
    pack = pack_ref[...][:, 0]
    lbl = jnp.where(sel_p, pack[:, 0, :], jnp.where(sel_n, 0.0, -1.0))
    i81 = lax.broadcasted_iota(jnp.int32, (_NUM_LABELS, b, tile), 0)
    lout_ref[...] = (i81.astype(jnp.float32) == lbl[None]).astype(jnp.float32)

    r = roi_ref[...][:, 0]
    w = r[:, 3, :] - r[:, 1, :]
    h = r[:, 2, :] - r[:, 0, :]
    cx = r[:, 1, :] + 0.5 * w
    cy = r[:, 0, :] + 0.5 * h
    gy1 = jnp.where(sel_p, pack[:, 1, :], 0.0)
    gx1 = jnp.where(sel_p, pack[:, 2, :], 0.0)
    gy2 = jnp.where(sel_p, pack[:, 3, :], 0.0)
    gx2 = jnp.where(sel_p, pack[:, 4, :], 0.0)
    gw = gx2 - gx1
    gh = gy2 - gy1
    gcx = gx1 + 0.5 * gw
    gcy = gy1 + 0.5 * gh
    ws = jnp.where(w == 0, 0.001, w)
    hs = jnp.where(h == 0, 0.001, h)
    gws = jnp.where(gw == 0, 1.0, gw)
    ghs = jnp.where(gh == 0, 1.0, gh)
    zero = jnp.zeros_like(gw)
    d_x = jnp.where(gw == 0, zero, (gcx - cx) / ws)
    d_y = jnp.where(gh == 0, zero, (gcy - cy) / hs)
    d_w = jnp.where(gw == 0, zero, jnp.log(gws / ws))
    d_h = jnp.where(gh == 0, zero, jnp.log(ghs / hs))
    jj = lax.broadcasted_iota(jnp.int32, (b, 1, 4, tile), 2)
    dval = jnp.where(jj == 0, d_y[:, None, None, :],
                     jnp.where(jj == 1, d_x[:, None, None, :],
                               jnp.where(jj == 2, d_h[:, None, None, :],
                                         d_w[:, None, None, :])))
    cc = lax.broadcasted_iota(jnp.int32, (b, _NUM_LABELS, 1, tile), 1)
    sel_cls = cc == lbl[:, None, None, :].astype(jnp.int32)
    dout_ref[...] = jnp.where(sel_cls, dval, 0.0)


def kernel(roi_bboxes, gt_boxes, gt_labels):
    b, n, _ = roi_bboxes.shape
    m = gt_boxes.shape[1]
    tile = _TILE
    np_, nt = _NP, _NT
    f32 = jnp.float32
    i32 = jnp.int32

    r_pos, r_neg = _make_priorities()
    roi_row = (jnp.pad(roi_bboxes, ((0, 0), (0, np_ - n), (0, 0)))
               .reshape(b, nt, tile, 4).transpose(0, 1, 3, 2))
    gv = jnp.concatenate(
        [gt_labels.astype(f32)[:, None, :], gt_boxes.transpose(0, 2, 1)],
        axis=1)

    mp, mn, pack = pl.pallas_call(
        _iou_gather_kernel,
        grid=(b, nt),
        in_specs=[
            pl.BlockSpec((1, 1, 4, tile), lambda bi, ki: (bi, ki, 0, 0)),
            pl.BlockSpec((1, m, 4), lambda bi, ki: (bi, 0, 0)),
            pl.BlockSpec((1, 5, m), lambda bi, ki: (bi, 0, 0)),
            pl.BlockSpec((1, 1, 1, tile), lambda bi, ki: (bi, ki, 0, 0)),
            pl.BlockSpec((1, 1, 1, tile), lambda bi, ki: (bi, ki, 0, 0)),
        ],
        out_specs=[
            pl.BlockSpec((1, 1, 1, tile), lambda bi, ki: (bi, ki, 0, 0)),
            pl.BlockSpec((1, 1, 1, tile), lambda bi, ki: (bi, ki, 0, 0)),
            pl.BlockSpec((1, 1, 5, tile), lambda bi, ki: (bi, ki, 0, 0)),
        ],
        out_shape=[
            jax.ShapeDtypeStruct((b, nt, 1, tile), i32),
            jax.ShapeDtypeStruct((b, nt, 1, tile), i32),
            jax.ShapeDtypeStruct((b, nt, 5, tile), f32),
        ],
        compiler_params=pltpu.CompilerParams(
            dimension_semantics=("parallel", "parallel")),
    )(roi_row, gt_boxes, gv, r_pos, r_neg)

    labels_t, deltas_t = pl.pallas_call(
        _assign_kernel,
        grid=(nt,),
        in_specs=[
            pl.BlockSpec((b, 1, 4, tile), lambda ki: (0, ki, 0, 0)),
            pl.BlockSpec((b, 1, 1, tile), lambda ki: (0, ki, 0, 0)),
            pl.BlockSpec((b, 1, 1, tile), lambda ki: (0, ki, 0, 0)),
            pl.BlockSpec((b, 1, 5, tile), lambda ki: (0, ki, 0, 0)),
            pl.BlockSpec((b, nt, 1, tile), lambda ki: (0, 0, 0, 0)),
            pl.BlockSpec((b, nt, 1, tile), lambda ki: (0, 0, 0, 0)),
        ],
        out_specs=[
            pl.BlockSpec((_NUM_LABELS, b, tile), lambda ki: (0, 0, ki)),
            pl.BlockSpec((b, _NUM_LABELS, 4, tile), lambda ki: (0, 0, 0, ki)),
        ],
        out_shape=[
            jax.ShapeDtypeStruct((_NUM_LABELS, b, n), f32),
            jax.ShapeDtypeStruct((b, _NUM_LABELS, 4, n), f32),
        ],
        scratch_shapes=[pltpu.SMEM((16,), i32)],
        compiler_params=pltpu.CompilerParams(
            dimension_semantics=("arbitrary",)),
    )(roi_row, mp, mn, pack, mp, mn)

    labels_out = jnp.transpose(labels_t, (1, 2, 0))
    deltas = jnp.transpose(deltas_t, (0, 3, 1, 2))
    return deltas, labels_out


# speedup vs baseline: 1.1955x; 1.1955x over previous
import jax
import jax.numpy as jnp
import numpy as np
from jax import lax
from jax.experimental import pallas as pl
from jax.experimental.pallas import tpu as pltpu

_NUM_LABELS = 81
_POS_K = 64
_NEG_K = 192
_TILE = 1024
_TILE_C = 1024
_B, _N, _M = 4, 20000, 100
_NP = ((_N + _TILE_C - 1) // _TILE_C) * _TILE_C
_NT = _NP // _TILE
_NT_C = _NP // _TILE_C


def _tf_block(key, x0, x1):
    rotations = ((13, 15, 26, 6), (17, 29, 16, 24))
    ks = (np.uint32(key[0]), np.uint32(key[1]),
          np.uint32(key[0]) ^ np.uint32(key[1]) ^ np.uint32(0x1BD11BDA))
    x0 = x0 + ks[0]
    x1 = x1 + ks[1]

    def rotl(x, d):
        return (x << np.uint32(d)) | (x >> np.uint32(32 - d))

    for i in range(5):
        for r in rotations[i % 2]:
            x0 = x0 + x1
            x1 = rotl(x1, r)
            x1 = x1 ^ x0
        x0 = x0 + ks[(i + 1) % 3]
        x1 = x1 + ks[(i + 2) % 3] + np.uint32(i + 1)
    return x0, x1


def _np_split(key):
    b1, b2 = _tf_block(key, np.zeros(2, np.uint32),
                       np.arange(2, dtype=np.uint32))
    return np.array([b1[0], b2[0]], np.uint32), np.array([b1[1], b2[1]],
                                                         np.uint32)


def _np_bits(key, size):
    b1, b2 = _tf_block(key, np.zeros(size, np.uint32),
                       np.arange(size, dtype=np.uint32))
    return b1 ^ b2


def _np_randint(key, size, maxval):
    k1, k2 = _np_split(key)
    higher = _np_bits(k1, size)
    lower = _np_bits(k2, size)
    span = np.uint32(maxval - 1)
    mult = np.uint32(2 ** 16) % span
    mult = (mult * mult) % span
    off = ((higher % span) * mult + lower % span) % span
    return (np.uint32(1) + off).astype(np.int32)


def _make_priorities():
    kp, kn = _np_split(np.array([0, 42], dtype=np.uint32))
    r_pos = _np_randint(kp, _B * _N, _POS_K * 10).reshape(_B, _N)
    r_neg = _np_randint(kn, _B * _N, _NEG_K * 10).reshape(_B, _N)
    pad = ((0, 0), (0, _NP - _N))
    return (np.pad(r_pos, pad).reshape(_B, _NT_C, 1, _TILE_C),
            np.pad(r_neg, pad).reshape(_B, _NT_C, 1, _TILE_C))


_R_POS, _R_NEG = _make_priorities()


def _iou_gather_kernel(roi_ref, gt_ref, gv_ref, rp_ref, rn_ref,
                       mp_ref, mn_ref, pack_ref):
    r = roi_ref[0, 0]
    by1 = r[0:1, :]
    bx1 = r[1:2, :]
    by2 = r[2:3, :]
    bx2 = r[3:4, :]
    g = gt_ref[0]
    gy1 = g[:, 0:1]
    gx1 = g[:, 1:2]
    gy2 = g[:, 2:3]
    gx2 = g[:, 3:4]
    x_top = jnp.maximum(bx1, gx1)
    y_top = jnp.maximum(by1, gy1)
    x_bot = jnp.minimum(bx2, gx2)
    y_bot = jnp.minimum(by2, gy2)
    inter = jnp.maximum(x_bot - x_top, 0.0) * jnp.maximum(y_bot - y_top, 0.0)
    barea = (by2 - by1) * (bx2 - bx1)
    iou = inter / (barea - inter)
    m = iou.shape[0]
    mx = jnp.max(iou, axis=0, keepdims=True)
    eqm = iou == mx
    i0 = lax.broadcasted_iota(jnp.int32, iou.shape, 0)
    am = jnp.min(jnp.where(eqm, i0, m), axis=0, keepdims=True)
    onehot = (i0 == am).astype(jnp.float32)
    pack = lax.dot_general(gv_ref[0], onehot, (((1,), (0,)), ((), ())),
                           precision=lax.Precision.HIGHEST,
                           preferred_element_type=jnp.float32)
    pos_c = mx > 0.5
    neg_c = jnp.logical_and(mx < 0.5, mx > 0.1)
    mp_ref[0, 0] = jnp.where(pos_c, rp_ref[0, 0], 0)
    mn_ref[0, 0] = jnp.where(neg_c, rn_ref[0, 0], 0)
    pack_ref[0, 0] = pack


def _select_thresholds(mfull, gidx, k_sel, hi0, n):
    def cnt_gt(t):
        return jnp.sum((mfull > t).astype(jnp.int32))

    def body_t(_, c):
        lo, hi = c
        mid = (lo + hi) // 2
        pred = cnt_gt(mid) < k_sel
        return (jnp.where(pred, lo, mid + 1), jnp.where(pred, mid, hi))

    lo, _ = lax.fori_loop(0, 11, body_t, (jnp.int32(0), jnp.int32(hi0)))
    t_val = lo
    need = k_sel - cnt_gt(t_val)
    eq = mfull == t_val

    def cnt_eq_lt(i):
        return jnp.sum(jnp.where(jnp.logical_and(eq, gidx < i), 1, 0))

    def body_i(_, c):
        lo2, hi2 = c
        mid = (lo2 + hi2) // 2
        pred = cnt_eq_lt(mid) >= need
        return (jnp.where(pred, lo2, mid + 1), jnp.where(pred, mid, hi2))

    lo2, _ = lax.fori_loop(0, 15, body_i, (jnp.int32(0), jnp.int32(n)))
    return t_val, lo2


def _assign_kernel(roi_ref, mp_ref, mn_ref, pack_ref, mpf_ref, mnf_ref,
                   lout_ref, dout_ref, sm_ref):
    k = pl.program_id(0)

    @pl.when(k == 0)
    def _():
        mpf = mpf_ref[...]
        mnf = mnf_ref[...]
        nb, nt, _, tile = mpf.shape
        n = nt * tile
        gidx = (lax.broadcasted_iota(jnp.int32, mpf.shape, 1) * tile
                + lax.broadcasted_iota(jnp.int32, mpf.shape, 3))
        for bi in range(nb):
            t_p, i_p = _select_thresholds(mpf[bi], gidx[bi], _POS_K,
                                          _POS_K * 10 - 1, n)
            t_n, i_n = _select_thresholds(mnf[bi], gidx[bi], _NEG_K,
                                          _NEG_K * 10 - 1, n)
            sm_ref[4 * bi + 0] = t_p
            sm_ref[4 * bi + 1] = i_p
            sm_ref[4 * bi + 2] = t_n
            sm_ref[4 * bi + 3] = i_n

    mp = mp_ref[...][:, 0, 0, :]
    mn = mn_ref[...][:, 0, 0, :]
    b, tile = mp.shape
    i32 = jnp.int32

    def col(off):
        return jnp.concatenate(
            [jnp.full((1, 1), sm_ref[4 * bi + off], i32) for bi in range(b)],
            axis=0)

    t_p = col(0)
    i_p = col(1)
    t_n = col(2)
    i_n = col(3)
    gidx = k * tile + lax.broadcasted_iota(jnp.int32, (b, tile), 1)
    sel_p = (mp > t_p) & (mp > 0) | ((mp == t_p) & (gidx < i_p) & (mp > 0))
    sel_n = (mn > t_n) & (mn > 0) | ((mn == t_n) & (gidx < i_n) & (mn > 0))
    pack = pack_ref[...][:, 0]
    lbl = jnp.where(sel_p, pack[:, 0, :], jnp.where(sel_n, 0.0, -1.0))
    i81 = lax.broadcasted_iota(jnp.int32, (_NUM_LABELS, b, tile), 0)
    lout_ref[...] = (i81.astype(jnp.float32) == lbl[None]).astype(jnp.float32)

    r = roi_ref[...][:, 0]
    w = r[:, 3, :] - r[:, 1, :]
    h = r[:, 2, :] - r[:, 0, :]
    cx = r[:, 1, :] + 0.5 * w
    cy = r[:, 0, :] + 0.5 * h
    gy1 = jnp.where(sel_p, pack[:, 1, :], 0.0)
    gx1 = jnp.where(sel_p, pack[:, 2, :], 0.0)
    gy2 = jnp.where(sel_p, pack[:, 3, :], 0.0)
    gx2 = jnp.where(sel_p, pack[:, 4, :], 0.0)
    gw = gx2 - gx1
    gh = gy2 - gy1
    gcx = gx1 + 0.5 * gw
    gcy = gy1 + 0.5 * gh
    ws = jnp.where(w == 0, 0.001, w)
    hs = jnp.where(h == 0, 0.001, h)
    gws = jnp.where(gw == 0, 1.0, gw)
    ghs = jnp.where(gh == 0, 1.0, gh)
    zero = jnp.zeros_like(gw)
    d_x = jnp.where(gw == 0, zero, (gcx - cx) / ws)
    d_y = jnp.where(gh == 0, zero, (gcy - cy) / hs)
    d_w = jnp.where(gw == 0, zero, jnp.log(gws / ws))
    d_h = jnp.where(gh == 0, zero, jnp.log(ghs / hs))
    jj = lax.broadcasted_iota(jnp.int32, (b, 1, 4, tile), 2)
    dval = jnp.where(jj == 0, d_y[:, None, None, :],
                     jnp.where(jj == 1, d_x[:, None, None, :],
                               jnp.where(jj == 2, d_h[:, None, None, :],
                                         d_w[:, None, None, :])))
    cc = lax.broadcasted_iota(jnp.int32, (b, _NUM_LABELS, 1, tile), 1)
    sel_cls = cc == lbl[:, None, None, :].astype(jnp.int32)
    dout_ref[...] = jnp.where(sel_cls, dval, 0.0)


def kernel(roi_bboxes, gt_boxes, gt_labels):
    b, n, _ = roi_bboxes.shape
    m = gt_boxes.shape[1]
    tile = _TILE
    tc = _TILE_C
    sub = tc // tile
    np_, nt, ntc = _NP, _NT, _NT_C
    f32 = jnp.float32
    i32 = jnp.int32

    r_pos, r_neg = _R_POS, _R_NEG
    roi_row = (jnp.pad(roi_bboxes, ((0, 0), (0, np_ - n), (0, 0)))
               .reshape(b, ntc, tc, 4).transpose(0, 1, 3, 2))
    gv = jnp.concatenate(
        [gt_labels.astype(f32)[:, None, :], gt_boxes.transpose(0, 2, 1)],
        axis=1)

    mp, mn, pack = pl.pallas_call(
        _iou_gather_kernel,
        grid=(b, nt),
        in_specs=[
            pl.BlockSpec((1, 1, 4, tile),
                         lambda bi, ki: (bi, ki // sub, 0, ki % sub)),
            pl.BlockSpec((1, m, 4), lambda bi, ki: (bi, 0, 0)),
            pl.BlockSpec((1, 5, m), lambda bi, ki: (bi, 0, 0)),
            pl.BlockSpec((1, 1, 1, tile),
                         lambda bi, ki: (bi, ki // sub, 0, ki % sub)),
            pl.BlockSpec((1, 1, 1, tile),
                         lambda bi, ki: (bi, ki // sub, 0, ki % sub)),
        ],
        out_specs=[
            pl.BlockSpec((1, 1, 1, tile),
                         lambda bi, ki: (bi, ki // sub, 0, ki % sub)),
            pl.BlockSpec((1, 1, 1, tile),
                         lambda bi, ki: (bi, ki // sub, 0, ki % sub)),
            pl.BlockSpec((1, 1, 5, tile),
                         lambda bi, ki: (bi, ki // sub, 0, ki % sub)),
        ],
        out_shape=[
            jax.ShapeDtypeStruct((b, ntc, 1, tc), i32),
            jax.ShapeDtypeStruct((b, ntc, 1, tc), i32),
            jax.ShapeDtypeStruct((b, ntc, 5, tc), f32),
        ],
        compiler_params=pltpu.CompilerParams(
            dimension_semantics=("parallel", "parallel")),
    )(roi_row, gt_boxes, gv, r_pos, r_neg)

    labels_t, deltas_t = pl.pallas_call(
        _assign_kernel,
        grid=(ntc,),
        in_specs=[
            pl.BlockSpec((b, 1, 4, tc), lambda ki: (0, ki, 0, 0)),
            pl.BlockSpec((b, 1, 1, tc), lambda ki: (0, ki, 0, 0)),
            pl.BlockSpec((b, 1, 1, tc), lambda ki: (0, ki, 0, 0)),
            pl.BlockSpec((b, 1, 5, tc), lambda ki: (0, ki, 0, 0)),
            pl.BlockSpec((b, ntc, 1, tc), lambda ki: (0, 0, 0, 0)),
            pl.BlockSpec((b, ntc, 1, tc), lambda ki: (0, 0, 0, 0)),
        ],
        out_specs=[
            pl.BlockSpec((_NUM_LABELS, b, tc), lambda ki: (0, 0, ki)),
            pl.BlockSpec((b, _NUM_LABELS, 4, tc), lambda ki: (0, 0, 0, ki)),
        ],
        out_shape=[
            jax.ShapeDtypeStruct((_NUM_LABELS, b, n), f32),
            jax.ShapeDtypeStruct((b, _NUM_LABELS, 4, n), f32),
        ],
        scratch_shapes=[pltpu.SMEM((16,), i32)],
        compiler_params=pltpu.CompilerParams(
            dimension_semantics=("arbitrary",)),
    )(roi_row, mp, mn, pack, mp, mn)

    labels_out = jnp.transpose(labels_t, (1, 2, 0))
    deltas = jnp.transpose(deltas_t, (0, 3, 1, 2))
    return deltas, labels_out


# speedup vs baseline: 1.3051x; 1.0916x over previous
import jax
import jax.numpy as jnp
import numpy as np
from jax import lax
from jax.experimental import pallas as pl
from jax.experimental.pallas import tpu as pltpu

_NUM_LABELS = 81
_POS_K = 64
_NEG_K = 192
_TILE = 2048
_TILE_C = 1024
_B, _N, _M = 4, 20000, 100
_NP = ((_N + _TILE - 1) // _TILE) * _TILE
_NT = _NP // _TILE
_NT_C = _NP // _TILE_C


def _tf_block(key, x0, x1):
    rotations = ((13, 15, 26, 6), (17, 29, 16, 24))
    ks = (np.uint32(key[0]), np.uint32(key[1]),
          np.uint32(key[0]) ^ np.uint32(key[1]) ^ np.uint32(0x1BD11BDA))
    x0 = x0 + ks[0]
    x1 = x1 + ks[1]

    def rotl(x, d):
        return (x << np.uint32(d)) | (x >> np.uint32(32 - d))

    for i in range(5):
        for r in rotations[i % 2]:
            x0 = x0 + x1
            x1 = rotl(x1, r)
            x1 = x1 ^ x0
        x0 = x0 + ks[(i + 1) % 3]
        x1 = x1 + ks[(i + 2) % 3] + np.uint32(i + 1)
    return x0, x1


def _np_split(key):
    b1, b2 = _tf_block(key, np.zeros(2, np.uint32),
                       np.arange(2, dtype=np.uint32))
    return np.array([b1[0], b2[0]], np.uint32), np.array([b1[1], b2[1]],
                                                         np.uint32)


def _np_bits(key, size):
    b1, b2 = _tf_block(key, np.zeros(size, np.uint32),
                       np.arange(size, dtype=np.uint32))
    return b1 ^ b2


def _np_randint(key, size, maxval):
    k1, k2 = _np_split(key)
    higher = _np_bits(k1, size)
    lower = _np_bits(k2, size)
    span = np.uint32(maxval - 1)
    mult = np.uint32(2 ** 16) % span
    mult = (mult * mult) % span
    off = ((higher % span) * mult + lower % span) % span
    return (np.uint32(1) + off).astype(np.int32)


def _make_priorities():
    kp, kn = _np_split(np.array([0, 42], dtype=np.uint32))
    r_pos = _np_randint(kp, _B * _N, _POS_K * 10).reshape(_B, _N)
    r_neg = _np_randint(kn, _B * _N, _NEG_K * 10).reshape(_B, _N)
    pad = ((0, 0), (0, _NP - _N))
    return (np.pad(r_pos, pad).reshape(_B, _NT, 1, _TILE),
            np.pad(r_neg, pad).reshape(_B, _NT, 1, _TILE))


_R_POS, _R_NEG = _make_priorities()


def _iou_gather_kernel(roi_ref, gt_ref, gv_ref, rp_ref, rn_ref,
                       mp_ref, mn_ref, pack_ref):
    r = roi_ref[0, 0]
    by1 = r[0:1, :]
    bx1 = r[1:2, :]
    by2 = r[2:3, :]
    bx2 = r[3:4, :]
    g = gt_ref[0]
    gy1 = g[:, 0:1]
    gx1 = g[:, 1:2]
    gy2 = g[:, 2:3]
    gx2 = g[:, 3:4]
    x_top = jnp.maximum(bx1, gx1)
    y_top = jnp.maximum(by1, gy1)
    x_bot = jnp.minimum(bx2, gx2)
    y_bot = jnp.minimum(by2, gy2)
    inter = jnp.maximum(x_bot - x_top, 0.0) * jnp.maximum(y_bot - y_top, 0.0)
    barea = (by2 - by1) * (bx2 - bx1)
    iou = inter / (barea - inter)
    m = iou.shape[0]
    mx = jnp.max(iou, axis=0, keepdims=True)
    eqm = iou == mx
    i0 = lax.broadcasted_iota(jnp.int32, iou.shape, 0)
    am = jnp.min(jnp.where(eqm, i0, m), axis=0, keepdims=True)
    onehot = (i0 == am).astype(jnp.float32)
    pack = lax.dot_general(gv_ref[0], onehot, (((1,), (0,)), ((), ())),
                           precision=lax.Precision.HIGHEST,
                           preferred_element_type=jnp.float32)
    pos_c = mx > 0.5
    neg_c = jnp.logical_and(mx < 0.5, mx > 0.1)
    mp_ref[0, 0] = jnp.where(pos_c, rp_ref[0, 0], 0)
    mn_ref[0, 0] = jnp.where(neg_c, rn_ref[0, 0], 0)
    pack_ref[0, 0] = pack


def _select_thresholds(mfull, gidx, k_sel, hi0, n):
    def cnt_gt(t):
        return jnp.sum((mfull > t).astype(jnp.int32))

    def body_t(_, c):
        lo, hi = c
        mid = (lo + hi) // 2
        pred = cnt_gt(mid) < k_sel
        return (jnp.where(pred, lo, mid + 1), jnp.where(pred, mid, hi))

    lo, _ = lax.fori_loop(0, 11, body_t, (jnp.int32(0), jnp.int32(hi0)))
    t_val = lo
    need = k_sel - cnt_gt(t_val)
    eq = mfull == t_val

    def cnt_eq_lt(i):
        return jnp.sum(jnp.where(jnp.logical_and(eq, gidx < i), 1, 0))

    def body_i(_, c):
        lo2, hi2 = c
        mid = (lo2 + hi2) // 2
        pred = cnt_eq_lt(mid) >= need
        return (jnp.where(pred, lo2, mid + 1), jnp.where(pred, mid, hi2))

    lo2, _ = lax.fori_loop(0, 15, body_i, (jnp.int32(0), jnp.int32(n)))
    return t_val, lo2


def _assign_kernel(roi_ref, mp_ref, mn_ref, pack_ref, mpf_ref, mnf_ref,
                   lout_ref, dout_ref, sm_ref):
    k = pl.program_id(0)

    @pl.when(k == 0)
    def _():
        mpf = mpf_ref[...]
        mnf = mnf_ref[...]
        nb, nt, _, tile = mpf.shape
        n = nt * tile
        gidx = (lax.broadcasted_iota(jnp.int32, mpf.shape, 1) * tile
                + lax.broadcasted_iota(jnp.int32, mpf.shape, 3))
        for bi in range(nb):
            t_p, i_p = _select_thresholds(mpf[bi], gidx[bi], _POS_K,
                                          _POS_K * 10 - 1, n)
            t_n, i_n = _select_thresholds(mnf[bi], gidx[bi], _NEG_K,
                                          _NEG_K * 10 - 1, n)
            sm_ref[4 * bi + 0] = t_p
            sm_ref[4 * bi + 1] = i_p
            sm_ref[4 * bi + 2] = t_n
            sm_ref[4 * bi + 3] = i_n

    mp = mp_ref[...][:, 0, 0, :]
    mn = mn_ref[...][:, 0, 0, :]
    b, tile = mp.shape
    i32 = jnp.int32

    def col(off):
        return jnp.concatenate(
            [jnp.full((1, 1), sm_ref[4 * bi + off], i32) for bi in range(b)],
            axis=0)

    t_p = col(0)
    i_p = col(1)
    t_n = col(2)
    i_n = col(3)
    gidx = k * tile + lax.broadcasted_iota(jnp.int32, (b, tile), 1)
    sel_p = (mp > t_p) & (mp > 0) | ((mp == t_p) & (gidx < i_p) & (mp > 0))
    sel_n = (mn > t_n) & (mn > 0) | ((mn == t_n) & (gidx < i_n) & (mn > 0))
    pack = pack_ref[...][:, 0]
    lbl = jnp.where(sel_p, pack[:, 0, :], jnp.where(sel_n, 0.0, -1.0))
    i81 = lax.broadcasted_iota(jnp.int32, (_NUM_LABELS, b, tile), 0)
    lout_ref[...] = (i81.astype(jnp.float32) == lbl[None]).astype(jnp.float32)

    r = roi_ref[...][:, 0]
    w = r[:, 3, :] - r[:, 1, :]
    h = r[:, 2, :] - r[:, 0, :]
    cx = r[:, 1, :] + 0.5 * w
    cy = r[:, 0, :] + 0.5 * h
    gy1 = jnp.where(sel_p, pack[:, 1, :], 0.0)
    gx1 = jnp.where(sel_p, pack[:, 2, :], 0.0)
    gy2 = jnp.where(sel_p, pack[:, 3, :], 0.0)
    gx2 = jnp.where(sel_p, pack[:, 4, :], 0.0)
    gw = gx2 - gx1
    gh = gy2 - gy1
    gcx = gx1 + 0.5 * gw
    gcy = gy1 + 0.5 * gh
    ws = jnp.where(w == 0, 0.001, w)
    hs = jnp.where(h == 0, 0.001, h)
    gws = jnp.where(gw == 0, 1.0, gw)
    ghs = jnp.where(gh == 0, 1.0, gh)
    zero = jnp.zeros_like(gw)
    d_x = jnp.where(gw == 0, zero, (gcx - cx) / ws)
    d_y = jnp.where(gh == 0, zero, (gcy - cy) / hs)
    d_w = jnp.where(gw == 0, zero, jnp.log(gws / ws))
    d_h = jnp.where(gh == 0, zero, jnp.log(ghs / hs))
    jj = lax.broadcasted_iota(jnp.int32, (b, 1, 4, tile), 2)
    dval = jnp.where(jj == 0, d_y[:, None, None, :],
                     jnp.where(jj == 1, d_x[:, None, None, :],
                               jnp.where(jj == 2, d_h[:, None, None, :],
                                         d_w[:, None, None, :])))
    cc = lax.broadcasted_iota(jnp.int32, (b, _NUM_LABELS, 1, tile), 1)
    sel_cls = cc == lbl[:, None, None, :].astype(jnp.int32)
    dout_ref[...] = jnp.where(sel_cls, dval, 0.0)


def kernel(roi_bboxes, gt_boxes, gt_labels):
    b, n, _ = roi_bboxes.shape
    m = gt_boxes.shape[1]
    tile = _TILE
    tc = _TILE_C
    sub = tile // tc
    np_, nt, ntc = _NP, _NT, _NT_C
    f32 = jnp.float32
    i32 = jnp.int32

    r_pos, r_neg = _R_POS, _R_NEG
    roi_row = (jnp.pad(roi_bboxes, ((0, 0), (0, np_ - n), (0, 0)))
               .reshape(b, nt, tile, 4).transpose(0, 1, 3, 2))
    gv = jnp.concatenate(
        [gt_labels.astype(f32)[:, None, :], gt_boxes.transpose(0, 2, 1)],
        axis=1)

    mp, mn, pack = pl.pallas_call(
        _iou_gather_kernel,
        grid=(b, nt),
        in_specs=[
            pl.BlockSpec((1, 1, 4, tile), lambda bi, ki: (bi, ki, 0, 0)),
            pl.BlockSpec((1, m, 4), lambda bi, ki: (bi, 0, 0)),
            pl.BlockSpec((1, 5, m), lambda bi, ki: (bi, 0, 0)),
            pl.BlockSpec((1, 1, 1, tile), lambda bi, ki: (bi, ki, 0, 0)),
            pl.BlockSpec((1, 1, 1, tile), lambda bi, ki: (bi, ki, 0, 0)),
        ],
        out_specs=[
            pl.BlockSpec((1, 1, 1, tile), lambda bi, ki: (bi, ki, 0, 0)),
            pl.BlockSpec((1, 1, 1, tile), lambda bi, ki: (bi, ki, 0, 0)),
            pl.BlockSpec((1, 1, 5, tile), lambda bi, ki: (bi, ki, 0, 0)),
        ],
        out_shape=[
            jax.ShapeDtypeStruct((b, nt, 1, tile), i32),
            jax.ShapeDtypeStruct((b, nt, 1, tile), i32),
            jax.ShapeDtypeStruct((b, nt, 5, tile), f32),
        ],
        compiler_params=pltpu.CompilerParams(
            dimension_semantics=("parallel", "parallel")),
    )(roi_row, gt_boxes, gv, r_pos, r_neg)

    labels_t, deltas_t = pl.pallas_call(
        _assign_kernel,
        grid=(ntc,),
        in_specs=[
            pl.BlockSpec((b, 1, 4, tc),
                         lambda ki: (0, ki // sub, 0, ki % sub)),
            pl.BlockSpec((b, 1, 1, tc),
                         lambda ki: (0, ki // sub, 0, ki % sub)),
            pl.BlockSpec((b, 1, 1, tc),
                         lambda ki: (0, ki // sub, 0, ki % sub)),
            pl.BlockSpec((b, 1, 5, tc),
                         lambda ki: (0, ki // sub, 0, ki % sub)),
            pl.BlockSpec((b, nt, 1, tile), lambda ki: (0, 0, 0, 0)),
            pl.BlockSpec((b, nt, 1, tile), lambda ki: (0, 0, 0, 0)),
        ],
        out_specs=[
            pl.BlockSpec((_NUM_LABELS, b, tc), lambda ki: (0, 0, ki)),
            pl.BlockSpec((b, _NUM_LABELS, 4, tc), lambda ki: (0, 0, 0, ki)),
        ],
        out_shape=[
            jax.ShapeDtypeStruct((_NUM_LABELS, b, n), f32),
            jax.ShapeDtypeStruct((b, _NUM_LABELS, 4, n), f32),
        ],
        scratch_shapes=[pltpu.SMEM((16,), i32)],
        compiler_params=pltpu.CompilerParams(
            dimension_semantics=("arbitrary",)),
    )(roi_row, mp, mn, pack, mp, mn)

    labels_out = jnp.transpose(labels_t, (1, 2, 0))
    deltas = jnp.transpose(deltas_t, (0, 3, 1, 2))
    return deltas, labels_out


# speedup vs baseline: 1.3371x; 1.0245x over previous
import jax
import jax.numpy as jnp
import numpy as np
from jax import lax
from jax.experimental import pallas as pl
from jax.experimental.pallas import tpu as pltpu

_NUM_LABELS = 81
_POS_K = 64
_NEG_K = 192
_TILE = 4096
_TILE_C = 1024
_B, _N, _M = 4, 20000, 100
_NP = ((_N + _TILE - 1) // _TILE) * _TILE
_NT = _NP // _TILE
_NT_C = _NP // _TILE_C


def _tf_block(key, x0, x1):
    rotations = ((13, 15, 26, 6), (17, 29, 16, 24))
    ks = (np.uint32(key[0]), np.uint32(key[1]),
          np.uint32(key[0]) ^ np.uint32(key[1]) ^ np.uint32(0x1BD11BDA))
    x0 = x0 + ks[0]
    x1 = x1 + ks[1]

    def rotl(x, d):
        return (x << np.uint32(d)) | (x >> np.uint32(32 - d))

    for i in range(5):
        for r in rotations[i % 2]:
            x0 = x0 + x1
            x1 = rotl(x1, r)
            x1 = x1 ^ x0
        x0 = x0 + ks[(i + 1) % 3]
        x1 = x1 + ks[(i + 2) % 3] + np.uint32(i + 1)
    return x0, x1


def _np_split(key):
    b1, b2 = _tf_block(key, np.zeros(2, np.uint32),
                       np.arange(2, dtype=np.uint32))
    return np.array([b1[0], b2[0]], np.uint32), np.array([b1[1], b2[1]],
                                                         np.uint32)


def _np_bits(key, size):
    b1, b2 = _tf_block(key, np.zeros(size, np.uint32),
                       np.arange(size, dtype=np.uint32))
    return b1 ^ b2


def _np_randint(key, size, maxval):
    k1, k2 = _np_split(key)
    higher = _np_bits(k1, size)
    lower = _np_bits(k2, size)
    span = np.uint32(maxval - 1)
    mult = np.uint32(2 ** 16) % span
    mult = (mult * mult) % span
    off = ((higher % span) * mult + lower % span) % span
    return (np.uint32(1) + off).astype(np.int32)


def _make_priorities():
    kp, kn = _np_split(np.array([0, 42], dtype=np.uint32))
    r_pos = _np_randint(kp, _B * _N, _POS_K * 10).reshape(_B, _N)
    r_neg = _np_randint(kn, _B * _N, _NEG_K * 10).reshape(_B, _N)
    pad = ((0, 0), (0, _NP - _N))
    return (np.pad(r_pos, pad).reshape(_B, _NT, 1, _TILE),
            np.pad(r_neg, pad).reshape(_B, _NT, 1, _TILE))


_R_POS, _R_NEG = _make_priorities()


def _iou_gather_kernel(roi_ref, gt_ref, gv_ref, rp_ref, rn_ref,
                       mp_ref, mn_ref, pack_ref):
    r = roi_ref[0, 0]
    by1 = r[0:1, :]
    bx1 = r[1:2, :]
    by2 = r[2:3, :]
    bx2 = r[3:4, :]
    g = gt_ref[0]
    gy1 = g[:, 0:1]
    gx1 = g[:, 1:2]
    gy2 = g[:, 2:3]
    gx2 = g[:, 3:4]
    x_top = jnp.maximum(bx1, gx1)
    y_top = jnp.maximum(by1, gy1)
    x_bot = jnp.minimum(bx2, gx2)
    y_bot = jnp.minimum(by2, gy2)
    inter = jnp.maximum(x_bot - x_top, 0.0) * jnp.maximum(y_bot - y_top, 0.0)
    barea = (by2 - by1) * (bx2 - bx1)
    iou = inter / (barea - inter)
    m = iou.shape[0]
    mx = jnp.max(iou, axis=0, keepdims=True)
    eqm = iou == mx
    i0 = lax.broadcasted_iota(jnp.int32, iou.shape, 0)
    am = jnp.min(jnp.where(eqm, i0, m), axis=0, keepdims=True)
    onehot = (i0 == am).astype(jnp.float32)
    pack = lax.dot_general(gv_ref[0], onehot, (((1,), (0,)), ((), ())),
                           precision=lax.Precision.HIGHEST,
                           preferred_element_type=jnp.float32)
    pos_c = mx > 0.5
    neg_c = jnp.logical_and(mx < 0.5, mx > 0.1)
    mp_ref[0, 0] = jnp.where(pos_c, rp_ref[0, 0], 0)
    mn_ref[0, 0] = jnp.where(neg_c, rn_ref[0, 0], 0)
    pack_ref[0, 0] = pack


def _select_thresholds(mfull, gidx, k_sel, hi0, n):
    def cnt_gt(t):
        return jnp.sum((mfull > t).astype(jnp.int32))

    def body_t(_, c):
        lo, hi = c
        mid = (lo + hi) // 2
        pred = cnt_gt(mid) < k_sel
        return (jnp.where(pred, lo, mid + 1), jnp.where(pred, mid, hi))

    lo, _ = lax.fori_loop(0, 11, body_t, (jnp.int32(0), jnp.int32(hi0)))
    t_val = lo
    need = k_sel - cnt_gt(t_val)
    eq = mfull == t_val

    def cnt_eq_lt(i):
        return jnp.sum(jnp.where(jnp.logical_and(eq, gidx < i), 1, 0))

    def body_i(_, c):
        lo2, hi2 = c
        mid = (lo2 + hi2) // 2
        pred = cnt_eq_lt(mid) >= need
        return (jnp.where(pred, lo2, mid + 1), jnp.where(pred, mid, hi2))

    lo2, _ = lax.fori_loop(0, 15, body_i, (jnp.int32(0), jnp.int32(n)))
    return t_val, lo2


def _assign_kernel(roi_ref, mp_ref, mn_ref, pack_ref, mpf_ref, mnf_ref,
                   lout_ref, dout_ref, sm_ref):
    k = pl.program_id(0)

    @pl.when(k == 0)
    def _():
        mpf = mpf_ref[...]
        mnf = mnf_ref[...]
        nb, nt, _, tile = mpf.shape
        n = nt * tile
        gidx = (lax.broadcasted_iota(jnp.int32, mpf.shape, 1) * tile
                + lax.broadcasted_iota(jnp.int32, mpf.shape, 3))
        for bi in range(nb):
            t_p, i_p = _select_thresholds(mpf[bi], gidx[bi], _POS_K,
                                          _POS_K * 10 - 1, n)
            t_n, i_n = _select_thresholds(mnf[bi], gidx[bi], _NEG_K,
                                          _NEG_K * 10 - 1, n)
            sm_ref[4 * bi + 0] = t_p
            sm_ref[4 * bi + 1] = i_p
            sm_ref[4 * bi + 2] = t_n
            sm_ref[4 * bi + 3] = i_n

    mp = mp_ref[...][:, 0, 0, :]
    mn = mn_ref[...][:, 0, 0, :]
    b, tile = mp.shape
    i32 = jnp.int32

    def col(off):
        return jnp.concatenate(
            [jnp.full((1, 1), sm_ref[4 * bi + off], i32) for bi in range(b)],
            axis=0)

    t_p = col(0)
    i_p = col(1)
    t_n = col(2)
    i_n = col(3)
    gidx = k * tile + lax.broadcasted_iota(jnp.int32, (b, tile), 1)
    sel_p = (mp > t_p) & (mp > 0) | ((mp == t_p) & (gidx < i_p) & (mp > 0))
    sel_n = (mn > t_n) & (mn > 0) | ((mn == t_n) & (gidx < i_n) & (mn > 0))
    pack = pack_ref[...][:, 0]
    lbl = jnp.where(sel_p, pack[:, 0, :], jnp.where(sel_n, 0.0, -1.0))
    i81 = lax.broadcasted_iota(jnp.int32, (_NUM_LABELS, b, tile), 0)
    lout_ref[...] = (i81.astype(jnp.float32) == lbl[None]).astype(jnp.float32)

    r = roi_ref[...][:, 0]
    w = r[:, 3, :] - r[:, 1, :]
    h = r[:, 2, :] - r[:, 0, :]
    cx = r[:, 1, :] + 0.5 * w
    cy = r[:, 0, :] + 0.5 * h
    gy1 = jnp.where(sel_p, pack[:, 1, :], 0.0)
    gx1 = jnp.where(sel_p, pack[:, 2, :], 0.0)
    gy2 = jnp.where(sel_p, pack[:, 3, :], 0.0)
    gx2 = jnp.where(sel_p, pack[:, 4, :], 0.0)
    gw = gx2 - gx1
    gh = gy2 - gy1
    gcx = gx1 + 0.5 * gw
    gcy = gy1 + 0.5 * gh
    ws = jnp.where(w == 0, 0.001, w)
    hs = jnp.where(h == 0, 0.001, h)
    gws = jnp.where(gw == 0, 1.0, gw)
    ghs = jnp.where(gh == 0, 1.0, gh)
    zero = jnp.zeros_like(gw)
    d_x = jnp.where(gw == 0, zero, (gcx - cx) / ws)
    d_y = jnp.where(gh == 0, zero, (gcy - cy) / hs)
    d_w = jnp.where(gw == 0, zero, jnp.log(gws / ws))
    d_h = jnp.where(gh == 0, zero, jnp.log(ghs / hs))
    jj = lax.broadcasted_iota(jnp.int32, (b, 1, 4, tile), 2)
    dval = jnp.where(jj == 0, d_y[:, None, None, :],
                     jnp.where(jj == 1, d_x[:, None, None, :],
                               jnp.where(jj == 2, d_h[:, None, None, :],
                                         d_w[:, None, None, :])))
    cc = lax.broadcasted_iota(jnp.int32, (b, _NUM_LABELS, 1, tile), 1)
    sel_cls = cc == lbl[:, None, None, :].astype(jnp.int32)
    dout_ref[...] = jnp.where(sel_cls, dval, 0.0)


def kernel(roi_bboxes, gt_boxes, gt_labels):
    b, n, _ = roi_bboxes.shape
    m = gt_boxes.shape[1]
    tile = _TILE
    tc = _TILE_C
    sub = tile // tc
    np_, nt, ntc = _NP, _NT, _NT_C
    f32 = jnp.float32
    i32 = jnp.int32

    r_pos, r_neg = _R_POS, _R_NEG
    roi_row = (jnp.pad(roi_bboxes, ((0, 0), (0, np_ - n), (0, 0)))
               .reshape(b, nt, tile, 4).transpose(0, 1, 3, 2))
    gv = jnp.concatenate(
        [gt_labels.astype(f32)[:, None, :], gt_boxes.transpose(0, 2, 1)],
        axis=1)

    mp, mn, pack = pl.pallas_call(
        _iou_gather_kernel,
        grid=(b, nt),
        in_specs=[
            pl.BlockSpec((1, 1, 4, tile), lambda bi, ki: (bi, ki, 0, 0)),
            pl.BlockSpec((1, m, 4), lambda bi, ki: (bi, 0, 0)),
            pl.BlockSpec((1, 5, m), lambda bi, ki: (bi, 0, 0)),
            pl.BlockSpec((1, 1, 1, tile), lambda bi, ki: (bi, ki, 0, 0)),
            pl.BlockSpec((1, 1, 1, tile), lambda bi, ki: (bi, ki, 0, 0)),
        ],
        out_specs=[
            pl.BlockSpec((1, 1, 1, tile), lambda bi, ki: (bi, ki, 0, 0)),
            pl.BlockSpec((1, 1, 1, tile), lambda bi, ki: (bi, ki, 0, 0)),
            pl.BlockSpec((1, 1, 5, tile), lambda bi, ki: (bi, ki, 0, 0)),
        ],
        out_shape=[
            jax.ShapeDtypeStruct((b, nt, 1, tile), i32),
            jax.ShapeDtypeStruct((b, nt, 1, tile), i32),
            jax.ShapeDtypeStruct((b, nt, 5, tile), f32),
        ],
        compiler_params=pltpu.CompilerParams(
            dimension_semantics=("parallel", "parallel")),
    )(roi_row, gt_boxes, gv, r_pos, r_neg)

    labels_t, deltas_t = pl.pallas_call(
        _assign_kernel,
        grid=(ntc,),
        in_specs=[
            pl.BlockSpec((b, 1, 4, tc),
                         lambda ki: (0, ki // sub, 0, ki % sub)),
            pl.BlockSpec((b, 1, 1, tc),
                         lambda ki: (0, ki // sub, 0, ki % sub)),
            pl.BlockSpec((b, 1, 1, tc),
                         lambda ki: (0, ki // sub, 0, ki % sub)),
            pl.BlockSpec((b, 1, 5, tc),
                         lambda ki: (0, ki // sub, 0, ki % sub)),
            pl.BlockSpec((b, nt, 1, tile), lambda ki: (0, 0, 0, 0)),
            pl.BlockSpec((b, nt, 1, tile), lambda ki: (0, 0, 0, 0)),
        ],
        out_specs=[
            pl.BlockSpec((_NUM_LABELS, b, tc), lambda ki: (0, 0, ki)),
            pl.BlockSpec((b, _NUM_LABELS, 4, tc), lambda ki: (0, 0, 0, ki)),
        ],
        out_shape=[
            jax.ShapeDtypeStruct((_NUM_LABELS, b, n), f32),
            jax.ShapeDtypeStruct((b, _NUM_LABELS, 4, n), f32),
        ],
        scratch_shapes=[pltpu.SMEM((16,), i32)],
        compiler_params=pltpu.CompilerParams(
            dimension_semantics=("arbitrary",)),
    )(roi_row, mp, mn, pack, mp, mn)

    labels_out = jnp.transpose(labels_t, (1, 2, 0))
    deltas = jnp.transpose(deltas_t, (0, 3, 1, 2))
    return deltas, labels_out
